# Initial kernel scaffold; baseline (speedup 1.0000x reference)
import jax
import jax.numpy as jnp
from jax.experimental import pallas as pl


def kernel(x, y, x_normals, y_normals):
    raise NotImplementedError("write your pallas kernel here")


# speedup vs baseline: 1.4926x; 1.4926x over previous
import jax
import jax.numpy as jnp
from jax import lax
from jax.experimental import pallas as pl

_TILE = 512
_EPS = 1e-6


def _chamfer_body(x_ref, y_ref, xn_ref, yn_ref, dist_ref, norm_ref):
    n = pl.program_id(0)
    nbatch = pl.num_programs(0)
    P1 = x_ref.shape[1]
    P2 = y_ref.shape[1]
    nt = P2 // _TILE

    x = x_ref[0]
    xn = xn_ref[0]
    x2 = jnp.sum(x * x, axis=1, keepdims=True)
    xh = xn / jnp.maximum(
        jnp.sqrt(jnp.sum(xn * xn, axis=1, keepdims=True)), _EPS)

    big = jnp.float32(3.4e38)

    def tile_step(t, carry):
        run_min, run_cos, s_cham_y, s_norm_y = carry
        yt = y_ref[0, pl.ds(t * _TILE, _TILE), :]
        ynt = yn_ref[0, pl.ds(t * _TILE, _TILE), :]
        y2t = jnp.sum(yt * yt, axis=1)[None, :]
        yh = ynt / jnp.maximum(
            jnp.sqrt(jnp.sum(ynt * ynt, axis=1, keepdims=True)), _EPS)

        xy = lax.dot_general(x, yt, (((1,), (1,)), ((), ())),
                             preferred_element_type=jnp.float32)
        d = x2 + y2t - 2.0 * xy
        c = lax.dot_general(xh, yh, (((1,), (1,)), ((), ())),
                            preferred_element_type=jnp.float32)

        iota_c = lax.broadcasted_iota(jnp.int32, (P1, _TILE), 1)
        iota_r = lax.broadcasted_iota(jnp.int32, (P1, _TILE), 0)

        tmin = jnp.min(d, axis=1, keepdims=True)
        tidx = jnp.min(jnp.where(d == tmin, iota_c, _TILE),
                       axis=1, keepdims=True)
        tcos = jnp.sum(jnp.where(iota_c == tidx, c, 0.0),
                       axis=1, keepdims=True)
        better = tmin < run_min
        run_min = jnp.where(better, tmin, run_min)
        run_cos = jnp.where(better, tcos, run_cos)

        cmin = jnp.min(d, axis=0, keepdims=True)
        cidx = jnp.min(jnp.where(d == cmin, iota_r, P1),
                       axis=0, keepdims=True)
        ccos = jnp.sum(jnp.where(iota_r == cidx, c, 0.0),
                       axis=0, keepdims=True)
        s_cham_y = s_cham_y + jnp.sum(cmin)
        s_norm_y = s_norm_y + jnp.sum(1.0 - jnp.abs(ccos))
        return run_min, run_cos, s_cham_y, s_norm_y

    init = (jnp.full((P1, 1), big, jnp.float32),
            jnp.zeros((P1, 1), jnp.float32),
            jnp.float32(0.0), jnp.float32(0.0))
    run_min, run_cos, s_cham_y, s_norm_y = lax.fori_loop(0, nt, tile_step, init)

    s_cham_x = jnp.sum(run_min)
    s_norm_x = jnp.sum(1.0 - jnp.abs(run_cos))

    d_contrib = (s_cham_x / P1 + s_cham_y / P2) / nbatch
    n_contrib = (s_norm_x / P1 + s_norm_y / P2) / nbatch

    @pl.when(n == 0)
    def _init():
        dist_ref[...] = jnp.zeros((1, 1), jnp.float32)
        norm_ref[...] = jnp.zeros((1, 1), jnp.float32)

    dist_ref[...] += d_contrib.reshape(1, 1)
    norm_ref[...] += n_contrib.reshape(1, 1)


def kernel(x, y, x_normals, y_normals):
    N, P1, D = x.shape
    dist, norm = pl.pallas_call(
        _chamfer_body,
        grid=(N,),
        in_specs=[pl.BlockSpec((1, P1, D), lambda n: (n, 0, 0))] * 4,
        out_specs=[pl.BlockSpec((1, 1), lambda n: (0, 0))] * 2,
        out_shape=[jax.ShapeDtypeStruct((1, 1), jnp.float32)] * 2,
    )(x, y, x_normals, y_normals)
    return (dist[0, 0], norm[0, 0])


# speedup vs baseline: 1.8198x; 1.2192x over previous
import jax
import jax.numpy as jnp
from jax import lax
from jax.experimental import pallas as pl
from jax.experimental.pallas import tpu as pltpu

_TILE = 512
_EPS = 1e-6


def _chamfer_body(x_ref, y_ref, xn_ref, yn_ref, dist_ref, norm_ref,
                  d_scr, rm_scr, cm_scr):
    n = pl.program_id(0)
    nbatch = pl.num_programs(0)
    P1 = x_ref.shape[1]
    P2 = y_ref.shape[1]
    nt = P2 // _TILE

    x = x_ref[0]
    xn = xn_ref[0]
    yn = yn_ref[0]
    x2 = jnp.sum(x * x, axis=1, keepdims=True)

    big = jnp.float32(3.4e38)
    rm_scr[...] = jnp.full((P1, 1), big, jnp.float32)

    def pass1(t, _):
        yt = y_ref[0, pl.ds(t * _TILE, _TILE), :]
        y2t = jnp.sum(yt * yt, axis=1)[None, :]
        xy = lax.dot_general(x, yt, (((1,), (1,)), ((), ())),
                             preferred_element_type=jnp.float32)
        d = x2 + y2t - 2.0 * xy
        d_scr[:, pl.ds(t * _TILE, _TILE)] = d
        rm_scr[...] = jnp.minimum(rm_scr[...], jnp.min(d, axis=1, keepdims=True))
        cm_scr[0, pl.ds(t * _TILE, _TILE)] = jnp.min(d, axis=0)
        return 0

    lax.fori_loop(0, nt, pass1, 0)

    rowmin = rm_scr[...]

    def pass2(t, carry):
        gx, s_norm_y = carry
        d = d_scr[:, pl.ds(t * _TILE, _TILE)]
        cmt = cm_scr[0, pl.ds(t * _TILE, _TILE)][None, :]
        maskx = jnp.where(d == rowmin, 1.0, 0.0)
        masky = jnp.where(d == cmt, 1.0, 0.0)
        ynt = yn_ref[0, pl.ds(t * _TILE, _TILE), :]
        gx = gx + lax.dot_general(maskx, ynt, (((1,), (0,)), ((), ())),
                                  preferred_element_type=jnp.float32)
        gy = lax.dot_general(masky, xn, (((0,), (0,)), ((), ())),
                             preferred_element_type=jnp.float32)
        ynt_n = jnp.maximum(jnp.sqrt(jnp.sum(ynt * ynt, axis=1)), _EPS)
        gy_n = jnp.maximum(jnp.sqrt(jnp.sum(gy * gy, axis=1)), _EPS)
        cos_y = jnp.sum(ynt * gy, axis=1) / (ynt_n * gy_n)
        s_norm_y = s_norm_y + jnp.sum(1.0 - jnp.abs(cos_y))
        return gx, s_norm_y

    gx, s_norm_y = lax.fori_loop(
        0, nt, pass2, (jnp.zeros((P1, 3), jnp.float32), jnp.float32(0.0)))

    xn_n = jnp.maximum(jnp.sqrt(jnp.sum(xn * xn, axis=1)), _EPS)
    gx_n = jnp.maximum(jnp.sqrt(jnp.sum(gx * gx, axis=1)), _EPS)
    cos_x = jnp.sum(xn * gx, axis=1) / (xn_n * gx_n)
    s_norm_x = jnp.sum(1.0 - jnp.abs(cos_x))

    s_cham_x = jnp.sum(rowmin)
    s_cham_y = jnp.sum(cm_scr[...])

    d_contrib = (s_cham_x / P1 + s_cham_y / P2) / nbatch
    n_contrib = (s_norm_x / P1 + s_norm_y / P2) / nbatch

    @pl.when(n == 0)
    def _init():
        dist_ref[...] = jnp.zeros((1, 1), jnp.float32)
        norm_ref[...] = jnp.zeros((1, 1), jnp.float32)

    dist_ref[...] += d_contrib.reshape(1, 1)
    norm_ref[...] += n_contrib.reshape(1, 1)


def kernel(x, y, x_normals, y_normals):
    N, P1, D = x.shape
    P2 = y.shape[1]
    dist, norm = pl.pallas_call(
        _chamfer_body,
        grid=(N,),
        in_specs=[pl.BlockSpec((1, P1, D), lambda n: (n, 0, 0))] * 4,
        out_specs=[pl.BlockSpec((1, 1), lambda n: (0, 0))] * 2,
        out_shape=[jax.ShapeDtypeStruct((1, 1), jnp.float32)] * 2,
        scratch_shapes=[
            pltpu.VMEM((P1, P2), jnp.float32),
            pltpu.VMEM((P1, 1), jnp.float32),
            pltpu.VMEM((1, P2), jnp.float32),
        ],
    )(x, y, x_normals, y_normals)
    return (dist[0, 0], norm[0, 0])
